# Initial kernel scaffold; baseline (speedup 1.0000x reference)
import jax
import jax.numpy as jnp
from jax.experimental import pallas as pl


def kernel(x, weight):
    raise NotImplementedError("write your pallas kernel here")


# speedup vs baseline: 4.8911x; 4.8911x over previous
import functools

import jax
import jax.numpy as jnp
from jax import lax
from jax.experimental import pallas as pl
from jax.experimental.pallas import tpu as pltpu
from jax.experimental.pallas import tpu_sc as plsc

VOCAB = 1000000
EMBED_DIM = 32
BATCH = 16384
HIST = 200

_NC = 2
_NS = 16
_NW = _NC * _NS
_TOTAL = BATCH * HIST
_PER_W = _TOTAL // _NW
_CHUNK = 512
_NBUF = 4
_NCHUNK = _PER_W // _CHUNK
_NGROUP = _NCHUNK // _NBUF

assert _PER_W * _NW == _TOTAL
assert _NCHUNK * _CHUNK == _PER_W
assert _NGROUP * _NBUF == _NCHUNK


def _embed_body(idx_hbm, tbl_hbm, out_hbm,
                idx0, idx1, idx2, idx3,
                rows0, rows1, rows2, rows3,
                sg0, sg1, sg2, sg3,
                so0, so1, so2, so3):
    idxs = [idx0, idx1, idx2, idx3]
    rows = [rows0, rows1, rows2, rows3]
    sg = [sg0, sg1, sg2, sg3]
    so = [so0, so1, so2, so3]

    wid = lax.axis_index("s") * _NC + lax.axis_index("c")
    base = wid * _PER_W

    for b in range(_NBUF):
        pltpu.sync_copy(idx_hbm.at[pl.ds(base + b * _CHUNK, _CHUNK)], idxs[b])
        pltpu.async_copy(tbl_hbm.at[idxs[b]], rows[b], sg[b])

    def steady(g, carry):
        for b in range(_NBUF):
            i = g * _NBUF + b
            off = base + i * _CHUNK
            pltpu.make_async_copy(tbl_hbm.at[idxs[b]], rows[b], sg[b]).wait()
            pltpu.async_copy(rows[b], out_hbm.at[pl.ds(off, _CHUNK)], so[b])
            pltpu.make_async_copy(
                rows[b], out_hbm.at[pl.ds(off, _CHUNK)], so[b]).wait()
            nxt = off + _NBUF * _CHUNK
            pltpu.sync_copy(idx_hbm.at[pl.ds(nxt, _CHUNK)], idxs[b])
            pltpu.async_copy(tbl_hbm.at[idxs[b]], rows[b], sg[b])
        return carry

    lax.fori_loop(0, _NGROUP - 1, steady, 0)

    for b in range(_NBUF):
        i = (_NGROUP - 1) * _NBUF + b
        off = base + i * _CHUNK
        pltpu.make_async_copy(tbl_hbm.at[idxs[b]], rows[b], sg[b]).wait()
        pltpu.async_copy(rows[b], out_hbm.at[pl.ds(off, _CHUNK)], so[b])
    for b in range(_NBUF):
        i = (_NGROUP - 1) * _NBUF + b
        off = base + i * _CHUNK
        pltpu.make_async_copy(
            rows[b], out_hbm.at[pl.ds(off, _CHUNK)], so[b]).wait()


@functools.partial(jax.jit, static_argnames=())
def _embed(idx, weight):
    fn = pl.kernel(
        _embed_body,
        mesh=plsc.VectorSubcoreMesh(core_axis_name="c", subcore_axis_name="s"),
        out_type=jax.ShapeDtypeStruct((_TOTAL, EMBED_DIM), jnp.float32),
        compiler_params=pltpu.CompilerParams(use_tc_tiling_on_sc=False),
        scratch_types=(
            [pltpu.VMEM((_CHUNK,), jnp.int32) for _ in range(_NBUF)]
            + [pltpu.VMEM((_CHUNK, EMBED_DIM), jnp.float32)
               for _ in range(_NBUF)]
            + [pltpu.SemaphoreType.DMA for _ in range(2 * _NBUF)]
        ),
    )
    return fn(idx, weight)


def kernel(x, weight):
    idx = x.reshape(-1).astype(jnp.int32)
    out = _embed(idx, weight)
    return out.reshape(BATCH, HIST, EMBED_DIM)
